# baseline (speedup 1.0000x reference)
import functools

import jax
import jax.numpy as jnp
from jax.experimental import pallas as pl
from jax.experimental.pallas import tpu as pltpu

N = 10000
E = 160000
F = 128
D = 128
C = 16
M = 50000
K = 8
HOPS = 3
RETRIEVE_W = 0.5
LABEL_W = 0.5

_TN = 1000
_TM = 2048
_MPAD = 51200


def _extract_topk(v, k):
    tn, w = v.shape
    neg = jnp.float32(-jnp.inf)
    ebits = (128 - jax.lax.broadcasted_iota(jnp.int32, (tn, w), 1)) << 23
    pow_row = jax.lax.bitcast_convert_type(ebits, jnp.float32)
    ones = jnp.ones((w, 1), jnp.float32)
    expmask = jnp.int32(-8388608)
    vals, sums = [], []
    x = v
    for _ in range(k):
        mv = jnp.max(x, axis=1, keepdims=True)
        mp = jnp.where(x == mv, pow_row, 0.0)
        s = jax.lax.dot_general(mp, ones, (((1,), (0,)), ((), ())),
                                preferred_element_type=jnp.float32)
        z = jax.lax.bitcast_convert_type(
            jax.lax.bitcast_convert_type(s, jnp.int32) & expmask, jnp.float32)
        vals.append(mv)
        sums.append(s)
        x = jnp.where(pow_row == z, neg, x)
    sall = jax.lax.bitcast_convert_type(
        jnp.concatenate(sums, axis=1), jnp.int32)
    pos = 128 - (sall >> 23)
    return jnp.concatenate(vals, axis=1), pos


def _knn_body(pre_ref, base_ref, vals_ref, idx_ref, *, k, tm):
    j = pl.program_id(1)
    sims = jax.lax.dot_general(
        pre_ref[...], base_ref[...],
        (((1,), (0,)), ((), ())),
        preferred_element_type=jnp.float32,
    )
    tn = sims.shape[0]
    nseg = min(128, tm)
    depth = tm // nseg
    slices = [sims[:, t * nseg:(t + 1) * nseg] for t in range(depth)]
    smax = slices[0]
    for t in range(1, depth):
        smax = jnp.maximum(smax, slices[t])
    _, spos = _extract_topk(smax, k)
    cand = jnp.concatenate(
        [jnp.take_along_axis(sl, spos, axis=1) for sl in slices], axis=1)
    tv, tlocal = _extract_topk(cand, k)
    sel = jnp.take_along_axis(spos.astype(jnp.float32), tlocal % k, axis=1)
    ti = sel + ((tlocal // k) * nseg + j * tm).astype(jnp.float32)

    @pl.when(j == 0)
    def _():
        vals_ref[...] = tv
        idx_ref[...] = ti

    @pl.when(j > 0)
    def _():
        wv = jnp.concatenate([vals_ref[...], tv], axis=1)
        wi = jnp.concatenate([idx_ref[...], ti], axis=1)
        nv, nlocal = _extract_topk(wv, k)
        vals_ref[...] = nv
        idx_ref[...] = jnp.take_along_axis(wi, nlocal, axis=1)


def _knn_topk(pre, base_emb_t, tn, tm, interpret=False):
    n, d = pre.shape
    m = base_emb_t.shape[1]
    return pl.pallas_call(
        functools.partial(_knn_body, k=K, tm=tm),
        grid=(n // tn, m // tm),
        in_specs=[
            pl.BlockSpec((tn, d), lambda i, j: (i, 0)),
            pl.BlockSpec((d, tm), lambda i, j: (0, j)),
        ],
        out_specs=[
            pl.BlockSpec((tn, K), lambda i, j: (i, 0)),
            pl.BlockSpec((tn, K), lambda i, j: (i, 0)),
        ],
        out_shape=[
            jax.ShapeDtypeStruct((n, K), jnp.float32),
            jax.ShapeDtypeStruct((n, K), jnp.float32),
        ],
        compiler_params=pltpu.CompilerParams(
            dimension_semantics=("parallel", "arbitrary")),
        interpret=interpret,
    )(pre, base_emb_t)


def kernel(features, edge_index, W_enc, base_emb, base_labels, W1, b1, W2, b2):
    src = edge_index[0]
    dst = edge_index[1]
    deg = jnp.clip(jnp.zeros((N,), dtype=jnp.float32).at[dst].add(1.0), 1.0, None)

    h = features @ W_enc
    pre = jax.nn.relu(
        jax.ops.segment_sum(h[src], dst, num_segments=N) / deg[:, None])

    base_pad_t = jnp.concatenate(
        [base_emb.T, jnp.zeros((D, _MPAD - M), dtype=base_emb.dtype)], axis=1)
    top_v, top_if = _knn_topk(pre, base_pad_t, _TN, _TM)
    top_i = top_if.astype(jnp.int32)
    w = jax.nn.softmax(top_v, axis=1)
    rag_embedding = jnp.einsum("nkd,nk->nd", jnp.take(base_emb, top_i, axis=0), w)
    rag_label = jnp.mean(jnp.take(base_labels, top_i, axis=0), axis=1)

    x = pre
    for _ in range(HOPS):
        x = jax.ops.segment_sum(x[src], dst, num_segments=N) / deg[:, None]

    hidden = x * (1.0 - RETRIEVE_W) + rag_embedding * RETRIEVE_W
    dec = jax.nn.relu(hidden @ W1 + b1) @ W2 + b2
    decode_label = jax.nn.softmax(dec, axis=1)
    return decode_label * (1.0 - LABEL_W) + rag_label * LABEL_W


# speedup vs baseline: 1.5383x; 1.5383x over previous
import functools

import jax
import jax.numpy as jnp
from jax.experimental import pallas as pl
from jax.experimental.pallas import tpu as pltpu

N = 10000
E = 160000
F = 128
D = 128
C = 16
M = 50000
K = 8
HOPS = 3
RETRIEVE_W = 0.5
LABEL_W = 0.5

_TN = 1000
_TM = 2048
_MPAD = 51200


def _extract_topk(v, k):
    tn, w = v.shape
    neg = jnp.float32(-jnp.inf)
    ebits = (128 - jax.lax.broadcasted_iota(jnp.int32, (tn, w), 1)) << 23
    pow_row = jax.lax.bitcast_convert_type(ebits, jnp.float32)
    vals, zs = [], []
    x = v
    for _ in range(k):
        mv = jnp.max(x, axis=1, keepdims=True)
        mp = jnp.where(x == mv, pow_row, 0.0)
        z = jnp.max(mp, axis=1, keepdims=True)
        vals.append(mv)
        zs.append(z)
        x = jnp.where(pow_row == z, neg, x)
    zall = jax.lax.bitcast_convert_type(
        jnp.concatenate(zs, axis=1), jnp.int32)
    pos = 128 - (zall >> 23)
    return jnp.concatenate(vals, axis=1), pos


def _knn_body(pre_ref, base_ref, vals_ref, idx_ref, *, k, tm):
    j = pl.program_id(1)
    sims = jax.lax.dot_general(
        pre_ref[...], base_ref[...],
        (((1,), (0,)), ((), ())),
        preferred_element_type=jnp.float32,
    )
    tn = sims.shape[0]
    nseg = min(128, tm)
    depth = tm // nseg
    slices = [sims[:, t * nseg:(t + 1) * nseg] for t in range(depth)]
    smax = slices[0]
    for t in range(1, depth):
        smax = jnp.maximum(smax, slices[t])
    _, spos = _extract_topk(smax, k)
    cand = jnp.concatenate(
        [jnp.take_along_axis(sl, spos, axis=1) for sl in slices], axis=1)
    tv, tlocal = _extract_topk(cand, k)
    sel = jnp.take_along_axis(spos.astype(jnp.float32), tlocal % k, axis=1)
    ti = sel + ((tlocal // k) * nseg + j * tm).astype(jnp.float32)

    @pl.when(j == 0)
    def _():
        vals_ref[...] = tv
        idx_ref[...] = ti

    @pl.when(j > 0)
    def _():
        wv = jnp.concatenate([vals_ref[...], tv], axis=1)
        wi = jnp.concatenate([idx_ref[...], ti], axis=1)
        nv, nlocal = _extract_topk(wv, k)
        vals_ref[...] = nv
        idx_ref[...] = jnp.take_along_axis(wi, nlocal, axis=1)


def _knn_topk(pre, base_emb_t, tn, tm, interpret=False):
    n, d = pre.shape
    m = base_emb_t.shape[1]
    return pl.pallas_call(
        functools.partial(_knn_body, k=K, tm=tm),
        grid=(n // tn, m // tm),
        in_specs=[
            pl.BlockSpec((tn, d), lambda i, j: (i, 0)),
            pl.BlockSpec((d, tm), lambda i, j: (0, j)),
        ],
        out_specs=[
            pl.BlockSpec((tn, K), lambda i, j: (i, 0)),
            pl.BlockSpec((tn, K), lambda i, j: (i, 0)),
        ],
        out_shape=[
            jax.ShapeDtypeStruct((n, K), jnp.float32),
            jax.ShapeDtypeStruct((n, K), jnp.float32),
        ],
        compiler_params=pltpu.CompilerParams(
            dimension_semantics=("parallel", "arbitrary")),
        interpret=interpret,
    )(pre, base_emb_t)


def kernel(features, edge_index, W_enc, base_emb, base_labels, W1, b1, W2, b2):
    src = edge_index[0]
    dst = edge_index[1]
    deg = jnp.clip(jnp.zeros((N,), dtype=jnp.float32).at[dst].add(1.0), 1.0, None)

    h = features @ W_enc
    pre = jax.nn.relu(
        jax.ops.segment_sum(h[src], dst, num_segments=N) / deg[:, None])

    base_pad_t = jnp.concatenate(
        [base_emb.T, jnp.zeros((D, _MPAD - M), dtype=base_emb.dtype)], axis=1)
    top_v, top_if = _knn_topk(pre, base_pad_t, _TN, _TM)
    top_i = top_if.astype(jnp.int32)
    w = jax.nn.softmax(top_v, axis=1)
    rag_embedding = jnp.einsum("nkd,nk->nd", jnp.take(base_emb, top_i, axis=0), w)
    rag_label = jnp.mean(jnp.take(base_labels, top_i, axis=0), axis=1)

    x = pre
    for _ in range(HOPS):
        x = jax.ops.segment_sum(x[src], dst, num_segments=N) / deg[:, None]

    hidden = x * (1.0 - RETRIEVE_W) + rag_embedding * RETRIEVE_W
    dec = jax.nn.relu(hidden @ W1 + b1) @ W2 + b2
    decode_label = jax.nn.softmax(dec, axis=1)
    return decode_label * (1.0 - LABEL_W) + rag_label * LABEL_W


# speedup vs baseline: 2.1466x; 1.3955x over previous
import functools

import jax
import jax.numpy as jnp
from jax import lax
from jax.experimental import pallas as pl
from jax.experimental.pallas import tpu as pltpu
from jax.experimental.pallas import tpu_sc as plsc

N = 10000
E = 160000
F = 128
D = 128
C = 16
M = 50000
K = 8
HOPS = 3
RETRIEVE_W = 0.5
LABEL_W = 0.5

_TN = 1000
_TM = 2048
_MPAD = 51200


def _extract_topk(v, k):
    tn, w = v.shape
    neg = jnp.float32(-jnp.inf)
    ebits = (128 - jax.lax.broadcasted_iota(jnp.int32, (tn, w), 1)) << 23
    pow_row = jax.lax.bitcast_convert_type(ebits, jnp.float32)
    vals, zs = [], []
    x = v
    for _ in range(k):
        mv = jnp.max(x, axis=1, keepdims=True)
        mp = jnp.where(x == mv, pow_row, 0.0)
        z = jnp.max(mp, axis=1, keepdims=True)
        vals.append(mv)
        zs.append(z)
        x = jnp.where(pow_row == z, neg, x)
    zall = jax.lax.bitcast_convert_type(
        jnp.concatenate(zs, axis=1), jnp.int32)
    pos = 128 - (zall >> 23)
    return jnp.concatenate(vals, axis=1), pos


def _knn_body(pre_ref, base_ref, vals_ref, idx_ref, *, k, tm):
    j = pl.program_id(1)
    sims = jax.lax.dot_general(
        pre_ref[...], base_ref[...],
        (((1,), (0,)), ((), ())),
        preferred_element_type=jnp.float32,
    )
    tn = sims.shape[0]
    nseg = min(128, tm)
    depth = tm // nseg
    slices = [sims[:, t * nseg:(t + 1) * nseg] for t in range(depth)]
    smax = slices[0]
    for t in range(1, depth):
        smax = jnp.maximum(smax, slices[t])
    _, spos = _extract_topk(smax, k)
    cand = jnp.concatenate(
        [jnp.take_along_axis(sl, spos, axis=1) for sl in slices], axis=1)
    tv, tlocal = _extract_topk(cand, k)
    sel = jnp.take_along_axis(spos.astype(jnp.float32), tlocal % k, axis=1)
    ti = sel + ((tlocal // k) * nseg + j * tm).astype(jnp.float32)

    @pl.when(j == 0)
    def _():
        vals_ref[...] = tv
        idx_ref[...] = ti

    @pl.when(j > 0)
    def _():
        wv = jnp.concatenate([vals_ref[...], tv], axis=1)
        wi = jnp.concatenate([idx_ref[...], ti], axis=1)
        nv, nlocal = _extract_topk(wv, k)
        vals_ref[...] = nv
        idx_ref[...] = jnp.take_along_axis(wi, nlocal, axis=1)


def _knn_topk(pre, base_emb_t, tn, tm, interpret=False):
    n, d = pre.shape
    m = base_emb_t.shape[1]
    return pl.pallas_call(
        functools.partial(_knn_body, k=K, tm=tm),
        grid=(n // tn, m // tm),
        in_specs=[
            pl.BlockSpec((tn, d), lambda i, j: (i, 0)),
            pl.BlockSpec((d, tm), lambda i, j: (0, j)),
        ],
        out_specs=[
            pl.BlockSpec((tn, K), lambda i, j: (i, 0)),
            pl.BlockSpec((tn, K), lambda i, j: (i, 0)),
        ],
        out_shape=[
            jax.ShapeDtypeStruct((n, K), jnp.float32),
            jax.ShapeDtypeStruct((n, K), jnp.float32),
        ],
        compiler_params=pltpu.CompilerParams(
            dimension_semantics=("parallel", "arbitrary")),
        interpret=interpret,
    )(pre, base_emb_t)


_SC_NC = 2
_SC_NS = 16
_SC_CH = 200
_SC_NPAD = 10240


def _sc_seg_sum(x, src, dst, zeros):
    nw = _SC_NC * _SC_NS
    epw = E // nw
    nch = epw // _SC_CH
    npad = _SC_NPAD
    rps = npad // _SC_NS
    mesh = plsc.VectorSubcoreMesh(core_axis_name="c", subcore_axis_name="s")

    @functools.partial(
        pl.kernel, mesh=mesh,
        out_type=jax.ShapeDtypeStruct((_SC_NC, npad, D), jnp.float32),
        scratch_types=[
            pltpu.VMEM((_SC_CH,), jnp.int32),
            pltpu.VMEM((_SC_CH,), jnp.int32),
            pltpu.VMEM((_SC_CH, D), jnp.float32),
            pltpu.VMEM_SHARED((npad, D), jnp.float32),
            pltpu.SemaphoreType.DMA,
        ])
    def k(x_hbm, src_hbm, dst_hbm, z_hbm, out_hbm, sidx, didx, rows, acc, sem):
        cid = lax.axis_index("c")
        sid = lax.axis_index("s")
        wid = sid * _SC_NC + cid
        pltpu.sync_copy(z_hbm.at[pl.ds(sid * rps, rps)],
                        acc.at[pl.ds(sid * rps, rps)])
        plsc.subcore_barrier()

        def body(c, carry):
            base = wid * epw + c * _SC_CH
            pltpu.sync_copy(src_hbm.at[pl.ds(base, _SC_CH)], sidx)
            pltpu.sync_copy(dst_hbm.at[pl.ds(base, _SC_CH)], didx)
            pltpu.async_copy(x_hbm.at[sidx], rows, sem).wait()
            pltpu.sync_copy(rows, acc.at[didx], add=True)
            return carry

        lax.fori_loop(0, nch, body, 0)
        plsc.subcore_barrier()
        pltpu.sync_copy(acc.at[pl.ds(sid * rps, rps)],
                        out_hbm.at[cid, pl.ds(sid * rps, rps)])

    return k(x, src, dst, zeros)


def kernel(features, edge_index, W_enc, base_emb, base_labels, W1, b1, W2, b2):
    src = edge_index[0].astype(jnp.int32)
    dst = edge_index[1].astype(jnp.int32)
    deg = jnp.clip(jnp.zeros((N,), dtype=jnp.float32).at[dst].add(1.0), 1.0, None)
    zeros = jnp.zeros((_SC_NPAD, D), dtype=jnp.float32)

    def hop(x):
        parts = _sc_seg_sum(x, src, dst, zeros)
        return (parts[0, :N] + parts[1, :N]) / deg[:, None]

    h = features @ W_enc
    pre = jax.nn.relu(hop(h))

    base_pad_t = jnp.concatenate(
        [base_emb.T, jnp.zeros((D, _MPAD - M), dtype=base_emb.dtype)], axis=1)
    top_v, top_if = _knn_topk(pre, base_pad_t, _TN, _TM)
    top_i = top_if.astype(jnp.int32)
    w = jax.nn.softmax(top_v, axis=1)
    rag_embedding = jnp.einsum("nkd,nk->nd", jnp.take(base_emb, top_i, axis=0), w)
    rag_label = jnp.mean(jnp.take(base_labels, top_i, axis=0), axis=1)

    x = pre
    for _ in range(HOPS):
        x = hop(x)

    hidden = x * (1.0 - RETRIEVE_W) + rag_embedding * RETRIEVE_W
    dec = jax.nn.relu(hidden @ W1 + b1) @ W2 + b2
    decode_label = jax.nn.softmax(dec, axis=1)
    return decode_label * (1.0 - LABEL_W) + rag_label * LABEL_W


# speedup vs baseline: 2.3155x; 1.0787x over previous
import functools

import jax
import jax.numpy as jnp
from jax import lax
from jax.experimental import pallas as pl
from jax.experimental.pallas import tpu as pltpu
from jax.experimental.pallas import tpu_sc as plsc

N = 10000
E = 160000
F = 128
D = 128
C = 16
M = 50000
K = 8
HOPS = 3
RETRIEVE_W = 0.5
LABEL_W = 0.5

_TN = 2000
_TM = 2048
_MPAD = 51200


def _extract_topk(v, k):
    tn, w = v.shape
    neg = jnp.float32(-jnp.inf)
    ebits = (128 - jax.lax.broadcasted_iota(jnp.int32, (tn, w), 1)) << 23
    pow_row = jax.lax.bitcast_convert_type(ebits, jnp.float32)
    vals, zs = [], []
    x = v
    for _ in range(k):
        mv = jnp.max(x, axis=1, keepdims=True)
        mp = jnp.where(x == mv, pow_row, 0.0)
        z = jnp.max(mp, axis=1, keepdims=True)
        vals.append(mv)
        zs.append(z)
        x = jnp.where(pow_row == z, neg, x)
    zall = jax.lax.bitcast_convert_type(
        jnp.concatenate(zs, axis=1), jnp.int32)
    pos = 128 - (zall >> 23)
    return jnp.concatenate(vals, axis=1), pos


def _knn_body(pre_ref, base_ref, vals_ref, idx_ref, *, k, tm):
    j = pl.program_id(1)
    sims = jax.lax.dot_general(
        pre_ref[...], base_ref[...],
        (((1,), (0,)), ((), ())),
        preferred_element_type=jnp.float32,
    )
    tn = sims.shape[0]
    nseg = min(128, tm)
    depth = tm // nseg
    slices = [sims[:, t * nseg:(t + 1) * nseg] for t in range(depth)]
    smax = slices[0]
    for t in range(1, depth):
        smax = jnp.maximum(smax, slices[t])
    _, spos = _extract_topk(smax, k)
    cand = jnp.concatenate(
        [jnp.take_along_axis(sl, spos, axis=1) for sl in slices], axis=1)
    tv, tlocal = _extract_topk(cand, k)
    sel = jnp.take_along_axis(spos.astype(jnp.float32), tlocal % k, axis=1)
    ti = sel + ((tlocal // k) * nseg + j * tm).astype(jnp.float32)

    @pl.when(j == 0)
    def _():
        vals_ref[...] = tv
        idx_ref[...] = ti

    @pl.when(j > 0)
    def _():
        wv = jnp.concatenate([vals_ref[...], tv], axis=1)
        wi = jnp.concatenate([idx_ref[...], ti], axis=1)
        nv, nlocal = _extract_topk(wv, k)
        vals_ref[...] = nv
        idx_ref[...] = jnp.take_along_axis(wi, nlocal, axis=1)


def _knn_topk(pre, base_emb_t, tn, tm, interpret=False):
    n, d = pre.shape
    m = base_emb_t.shape[1]
    return pl.pallas_call(
        functools.partial(_knn_body, k=K, tm=tm),
        grid=(n // tn, m // tm),
        in_specs=[
            pl.BlockSpec((tn, d), lambda i, j: (i, 0)),
            pl.BlockSpec((d, tm), lambda i, j: (0, j)),
        ],
        out_specs=[
            pl.BlockSpec((tn, K), lambda i, j: (i, 0)),
            pl.BlockSpec((tn, K), lambda i, j: (i, 0)),
        ],
        out_shape=[
            jax.ShapeDtypeStruct((n, K), jnp.float32),
            jax.ShapeDtypeStruct((n, K), jnp.float32),
        ],
        compiler_params=pltpu.CompilerParams(
            dimension_semantics=("parallel", "arbitrary")),
        interpret=interpret,
    )(pre, base_emb_t)


_SC_NC = 2
_SC_NS = 16
_SC_CH = 200
_SC_NPAD = 10240


def _sc_seg_sum(x, src, dst, zeros):
    nw = _SC_NC * _SC_NS
    epw = E // nw
    nch = epw // _SC_CH
    npad = _SC_NPAD
    rps = npad // _SC_NS
    mesh = plsc.VectorSubcoreMesh(core_axis_name="c", subcore_axis_name="s")

    @functools.partial(
        pl.kernel, mesh=mesh,
        out_type=jax.ShapeDtypeStruct((_SC_NC, npad, D), jnp.float32),
        scratch_types=[
            pltpu.VMEM((_SC_CH,), jnp.int32),
            pltpu.VMEM((_SC_CH,), jnp.int32),
            pltpu.VMEM((_SC_CH, D), jnp.float32),
            pltpu.VMEM_SHARED((npad, D), jnp.float32),
            pltpu.SemaphoreType.DMA,
        ])
    def k(x_hbm, src_hbm, dst_hbm, z_hbm, out_hbm, sidx, didx, rows, acc, sem):
        cid = lax.axis_index("c")
        sid = lax.axis_index("s")
        wid = sid * _SC_NC + cid
        pltpu.sync_copy(z_hbm.at[pl.ds(sid * rps, rps)],
                        acc.at[pl.ds(sid * rps, rps)])
        plsc.subcore_barrier()

        def body(c, carry):
            base = wid * epw + c * _SC_CH
            pltpu.sync_copy(src_hbm.at[pl.ds(base, _SC_CH)], sidx)
            pltpu.sync_copy(dst_hbm.at[pl.ds(base, _SC_CH)], didx)
            pltpu.async_copy(x_hbm.at[sidx], rows, sem).wait()
            pltpu.sync_copy(rows, acc.at[didx], add=True)
            return carry

        lax.fori_loop(0, nch, body, 0)
        plsc.subcore_barrier()
        pltpu.sync_copy(acc.at[pl.ds(sid * rps, rps)],
                        out_hbm.at[cid, pl.ds(sid * rps, rps)])

    return k(x, src, dst, zeros)


def kernel(features, edge_index, W_enc, base_emb, base_labels, W1, b1, W2, b2):
    src = edge_index[0].astype(jnp.int32)
    dst = edge_index[1].astype(jnp.int32)
    zeros = jnp.zeros((_SC_NPAD, D), dtype=jnp.float32)

    deg = jnp.clip(jnp.zeros((N,), dtype=jnp.float32).at[dst].add(1.0), 1.0, None)

    def hop(x):
        p = _sc_seg_sum(x, src, dst, zeros)
        return (p[0, :N] + p[1, :N]) / deg[:, None]

    h = features @ W_enc
    pre = jax.nn.relu(hop(h))

    base_pad_t = jnp.concatenate(
        [base_emb.T, jnp.zeros((D, _MPAD - M), dtype=base_emb.dtype)], axis=1)
    top_v, top_if = _knn_topk(pre, base_pad_t, _TN, _TM)
    top_i = top_if.astype(jnp.int32)
    w = jax.nn.softmax(top_v, axis=1)
    rag_embedding = jnp.einsum("nkd,nk->nd", jnp.take(base_emb, top_i, axis=0), w)
    rag_label = jnp.mean(jnp.take(base_labels, top_i, axis=0), axis=1)

    x = pre
    for _ in range(HOPS):
        x = hop(x)

    hidden = x * (1.0 - RETRIEVE_W) + rag_embedding * RETRIEVE_W
    dec = jax.nn.relu(hidden @ W1 + b1) @ W2 + b2
    decode_label = jax.nn.softmax(dec, axis=1)
    return decode_label * (1.0 - LABEL_W) + rag_label * LABEL_W


# speedup vs baseline: 2.4079x; 1.0399x over previous
import functools

import jax
import jax.numpy as jnp
from jax import lax
from jax.experimental import pallas as pl
from jax.experimental.pallas import tpu as pltpu
from jax.experimental.pallas import tpu_sc as plsc

N = 10000
E = 160000
F = 128
D = 128
C = 16
M = 50000
K = 8
HOPS = 3
RETRIEVE_W = 0.5
LABEL_W = 0.5

_TN = 2000
_TM = 2048
_MPAD = 51200


def _extract_topk(v, k):
    tn, w = v.shape
    neg = jnp.float32(-jnp.inf)
    ebits = (128 - jax.lax.broadcasted_iota(jnp.int32, (tn, w), 1)) << 23
    pow_row = jax.lax.bitcast_convert_type(ebits, jnp.float32)
    vals, zs = [], []
    x = v
    for _ in range(k):
        mv = jnp.max(x, axis=1, keepdims=True)
        mp = jnp.where(x == mv, pow_row, 0.0)
        z = jnp.max(mp, axis=1, keepdims=True)
        vals.append(mv)
        zs.append(z)
        x = jnp.where(pow_row == z, neg, x)
    zall = jax.lax.bitcast_convert_type(
        jnp.concatenate(zs, axis=1), jnp.int32)
    pos = 128 - (zall >> 23)
    return jnp.concatenate(vals, axis=1), pos


def _knn_body(pre_ref, base_ref, vals_ref, idx_ref, *, k, tm):
    j = pl.program_id(1)
    sims = jax.lax.dot_general(
        pre_ref[...], base_ref[...],
        (((1,), (0,)), ((), ())),
        preferred_element_type=jnp.float32,
    )
    tn = sims.shape[0]
    nseg = min(128, tm)
    depth = tm // nseg
    slices = [sims[:, t * nseg:(t + 1) * nseg] for t in range(depth)]
    smax = slices[0]
    for t in range(1, depth):
        smax = jnp.maximum(smax, slices[t])
    _, spos = _extract_topk(smax, k)
    cand = jnp.concatenate(
        [jnp.take_along_axis(sl, spos, axis=1) for sl in slices], axis=1)
    tv, tlocal = _extract_topk(cand, k)
    sel = jnp.take_along_axis(spos.astype(jnp.float32), tlocal % k, axis=1)
    ti = sel + ((tlocal // k) * nseg + j * tm).astype(jnp.float32)

    @pl.when(j == 0)
    def _():
        vals_ref[...] = tv
        idx_ref[...] = ti

    @pl.when(j > 0)
    def _():
        wv = jnp.concatenate([vals_ref[...], tv], axis=1)
        wi = jnp.concatenate([idx_ref[...], ti], axis=1)
        nv, nlocal = _extract_topk(wv, k)
        vals_ref[...] = nv
        idx_ref[...] = jnp.take_along_axis(wi, nlocal, axis=1)


def _knn_topk(pre, base_emb_t, tn, tm, interpret=False):
    n, d = pre.shape
    m = base_emb_t.shape[1]
    return pl.pallas_call(
        functools.partial(_knn_body, k=K, tm=tm),
        grid=(n // tn, m // tm),
        in_specs=[
            pl.BlockSpec((tn, d), lambda i, j: (i, 0)),
            pl.BlockSpec((d, tm), lambda i, j: (0, j)),
        ],
        out_specs=[
            pl.BlockSpec((tn, K), lambda i, j: (i, 0)),
            pl.BlockSpec((tn, K), lambda i, j: (i, 0)),
        ],
        out_shape=[
            jax.ShapeDtypeStruct((n, K), jnp.float32),
            jax.ShapeDtypeStruct((n, K), jnp.float32),
        ],
        compiler_params=pltpu.CompilerParams(
            dimension_semantics=("parallel", "arbitrary")),
        interpret=interpret,
    )(pre, base_emb_t)


_SC_NC = 2
_SC_NS = 16
_SC_CH = 200
_SC_NPAD = 10240


def _sc_seg_sum(x, src, dst, zeros):
    nw = _SC_NC * _SC_NS
    epw = E // nw
    nch = epw // _SC_CH
    npad = _SC_NPAD
    rps = npad // _SC_NS
    mesh = plsc.VectorSubcoreMesh(core_axis_name="c", subcore_axis_name="s")

    @functools.partial(
        pl.kernel, mesh=mesh,
        out_type=jax.ShapeDtypeStruct((_SC_NC, npad, D), jnp.float32),
        scratch_types=[
            pltpu.VMEM((_SC_CH,), jnp.int32),
            pltpu.VMEM((_SC_CH,), jnp.int32),
            pltpu.VMEM((_SC_CH, D), jnp.float32),
            pltpu.VMEM_SHARED((npad, D), jnp.float32),
            pltpu.SemaphoreType.DMA,
        ])
    def k(x_hbm, src_hbm, dst_hbm, z_hbm, out_hbm, sidx, didx, rows, acc, sem):
        cid = lax.axis_index("c")
        sid = lax.axis_index("s")
        wid = sid * _SC_NC + cid
        pltpu.sync_copy(z_hbm.at[pl.ds(sid * rps, rps)],
                        acc.at[pl.ds(sid * rps, rps)])
        plsc.subcore_barrier()

        def body(c, carry):
            base = wid * epw + c * _SC_CH
            pltpu.sync_copy(src_hbm.at[pl.ds(base, _SC_CH)], sidx)
            pltpu.sync_copy(dst_hbm.at[pl.ds(base, _SC_CH)], didx)
            pltpu.async_copy(x_hbm.at[sidx], rows, sem).wait()
            pltpu.sync_copy(rows, acc.at[didx], add=True)
            return carry

        lax.fori_loop(0, nch, body, 0)
        plsc.subcore_barrier()
        pltpu.sync_copy(acc.at[pl.ds(sid * rps, rps)],
                        out_hbm.at[cid, pl.ds(sid * rps, rps)])

    return k(x, src, dst, zeros)


def _sc_degree(dst, zeros, ones):
    nw = _SC_NC * _SC_NS
    epw = E // nw
    nch = epw // _SC_CH
    npad = _SC_NPAD
    rps = npad // _SC_NS
    mesh = plsc.VectorSubcoreMesh(core_axis_name="c", subcore_axis_name="s")

    @functools.partial(
        pl.kernel, mesh=mesh,
        out_type=jax.ShapeDtypeStruct((_SC_NC, npad, D), jnp.float32),
        scratch_types=[
            pltpu.VMEM((_SC_CH,), jnp.int32),
            pltpu.VMEM((_SC_CH, D), jnp.float32),
            pltpu.VMEM_SHARED((npad, D), jnp.float32),
        ])
    def k(dst_hbm, z_hbm, o_hbm, out_hbm, didx, ones_v, acc):
        cid = lax.axis_index("c")
        sid = lax.axis_index("s")
        wid = sid * _SC_NC + cid
        pltpu.sync_copy(z_hbm.at[pl.ds(sid * rps, rps)],
                        acc.at[pl.ds(sid * rps, rps)])
        pltpu.sync_copy(o_hbm, ones_v)
        plsc.subcore_barrier()

        def body(c, carry):
            base = wid * epw + c * _SC_CH
            pltpu.sync_copy(dst_hbm.at[pl.ds(base, _SC_CH)], didx)
            pltpu.sync_copy(ones_v, acc.at[didx], add=True)
            return carry

        lax.fori_loop(0, nch, body, 0)
        plsc.subcore_barrier()
        pltpu.sync_copy(acc.at[pl.ds(sid * rps, rps)],
                        out_hbm.at[cid, pl.ds(sid * rps, rps)])

    return k(dst, zeros, ones)


def kernel(features, edge_index, W_enc, base_emb, base_labels, W1, b1, W2, b2):
    src = edge_index[0].astype(jnp.int32)
    dst = edge_index[1].astype(jnp.int32)
    zeros = jnp.zeros((_SC_NPAD, D), dtype=jnp.float32)
    dparts = _sc_degree(dst, zeros, jnp.ones((_SC_CH, D), dtype=jnp.float32))
    deg = jnp.clip(dparts[0, :N, 0] + dparts[1, :N, 0], 1.0, None)

    def hop(x):
        p = _sc_seg_sum(x, src, dst, zeros)
        return (p[0, :N] + p[1, :N]) / deg[:, None]

    h = features @ W_enc
    pre = jax.nn.relu(hop(h))

    base_pad_t = jnp.concatenate(
        [base_emb.T, jnp.zeros((D, _MPAD - M), dtype=base_emb.dtype)], axis=1)
    top_v, top_if = _knn_topk(pre, base_pad_t, _TN, _TM)
    top_i = top_if.astype(jnp.int32)
    w = jax.nn.softmax(top_v, axis=1)
    rag_embedding = jnp.einsum("nkd,nk->nd", jnp.take(base_emb, top_i, axis=0), w)
    rag_label = jnp.mean(jnp.take(base_labels, top_i, axis=0), axis=1)

    x = pre
    for _ in range(HOPS):
        x = hop(x)

    hidden = x * (1.0 - RETRIEVE_W) + rag_embedding * RETRIEVE_W
    dec = jax.nn.relu(hidden @ W1 + b1) @ W2 + b2
    decode_label = jax.nn.softmax(dec, axis=1)
    return decode_label * (1.0 - LABEL_W) + rag_label * LABEL_W


# speedup vs baseline: 2.4084x; 1.0002x over previous
import functools

import jax
import jax.numpy as jnp
from jax import lax
from jax.experimental import pallas as pl
from jax.experimental.pallas import tpu as pltpu
from jax.experimental.pallas import tpu_sc as plsc

N = 10000
E = 160000
F = 128
D = 128
C = 16
M = 50000
K = 8
HOPS = 3
RETRIEVE_W = 0.5
LABEL_W = 0.5

_TN = 2000
_TM = 2048
_MPAD = 51200


def _extract_topk(v, k):
    tn, w = v.shape
    neg = jnp.float32(-jnp.inf)
    ebits = (128 - jax.lax.broadcasted_iota(jnp.int32, (tn, w), 1)) << 23
    pow_row = jax.lax.bitcast_convert_type(ebits, jnp.float32)
    vals, zs = [], []
    x = v
    for _ in range(k):
        mv = jnp.max(x, axis=1, keepdims=True)
        mp = jnp.where(x == mv, pow_row, 0.0)
        z = jnp.max(mp, axis=1, keepdims=True)
        vals.append(mv)
        zs.append(z)
        x = jnp.where(pow_row == z, neg, x)
    zall = jax.lax.bitcast_convert_type(
        jnp.concatenate(zs, axis=1), jnp.int32)
    pos = 128 - (zall >> 23)
    return jnp.concatenate(vals, axis=1), pos


def _knn_body(pre_ref, base_ref, vals_ref, idx_ref, *, k, tm):
    j = pl.program_id(1)
    sims = jax.lax.dot_general(
        pre_ref[...], base_ref[...],
        (((1,), (0,)), ((), ())),
        preferred_element_type=jnp.float32,
    )
    tn = sims.shape[0]
    nseg = min(128, tm)
    depth = tm // nseg
    slices = [sims[:, t * nseg:(t + 1) * nseg] for t in range(depth)]
    smax = slices[0]
    for t in range(1, depth):
        smax = jnp.maximum(smax, slices[t])
    _, spos = _extract_topk(smax, k)
    cand = jnp.concatenate(
        [jnp.take_along_axis(sl, spos, axis=1) for sl in slices], axis=1)
    tv, tlocal = _extract_topk(cand, k)
    sel = jnp.take_along_axis(spos.astype(jnp.float32), tlocal % k, axis=1)
    ti = sel + ((tlocal // k) * nseg + j * tm).astype(jnp.float32)

    @pl.when(j == 0)
    def _():
        vals_ref[...] = tv
        idx_ref[...] = ti

    @pl.when(j > 0)
    def _():
        wv = jnp.concatenate([vals_ref[...], tv], axis=1)
        wi = jnp.concatenate([idx_ref[...], ti], axis=1)
        nv, nlocal = _extract_topk(wv, k)
        vals_ref[...] = nv
        idx_ref[...] = jnp.take_along_axis(wi, nlocal, axis=1)


def _knn_topk(pre, base_emb_t, tn, tm):
    n, d = pre.shape
    m = base_emb_t.shape[1]
    return pl.pallas_call(
        functools.partial(_knn_body, k=K, tm=tm),
        grid=(n // tn, m // tm),
        in_specs=[
            pl.BlockSpec((tn, d), lambda i, j: (i, 0)),
            pl.BlockSpec((d, tm), lambda i, j: (0, j)),
        ],
        out_specs=[
            pl.BlockSpec((tn, K), lambda i, j: (i, 0)),
            pl.BlockSpec((tn, K), lambda i, j: (i, 0)),
        ],
        out_shape=[
            jax.ShapeDtypeStruct((n, K), jnp.float32),
            jax.ShapeDtypeStruct((n, K), jnp.float32),
        ],
        compiler_params=pltpu.CompilerParams(
            dimension_semantics=("parallel", "arbitrary")),
    )(pre, base_emb_t)


_SC_NC = 2
_SC_NS = 16
_SC_CH = 200
_SC_NPAD = 10240


def _sc_seg_sum(x, src, dst, zeros):
    nw = _SC_NC * _SC_NS
    epw = E // nw
    nch = epw // _SC_CH
    npad = _SC_NPAD
    rps = npad // _SC_NS
    mesh = plsc.VectorSubcoreMesh(core_axis_name="c", subcore_axis_name="s")

    @functools.partial(
        pl.kernel, mesh=mesh,
        out_type=jax.ShapeDtypeStruct((_SC_NC, npad, D), jnp.float32),
        scratch_types=[
            pltpu.VMEM((_SC_CH,), jnp.int32),
            pltpu.VMEM((_SC_CH,), jnp.int32),
            pltpu.VMEM((_SC_CH, D), jnp.float32),
            pltpu.VMEM_SHARED((npad, D), jnp.float32),
            pltpu.SemaphoreType.DMA,
        ])
    def k(x_hbm, src_hbm, dst_hbm, z_hbm, out_hbm, sidx, didx, rows, acc, sem):
        cid = lax.axis_index("c")
        sid = lax.axis_index("s")
        wid = sid * _SC_NC + cid
        pltpu.sync_copy(z_hbm.at[pl.ds(sid * rps, rps)],
                        acc.at[pl.ds(sid * rps, rps)])
        plsc.subcore_barrier()

        def body(c, carry):
            base = wid * epw + c * _SC_CH
            pltpu.sync_copy(src_hbm.at[pl.ds(base, _SC_CH)], sidx)
            pltpu.sync_copy(dst_hbm.at[pl.ds(base, _SC_CH)], didx)
            pltpu.async_copy(x_hbm.at[sidx], rows, sem).wait()
            pltpu.sync_copy(rows, acc.at[didx], add=True)
            return carry

        lax.fori_loop(0, nch, body, 0)
        plsc.subcore_barrier()
        pltpu.sync_copy(acc.at[pl.ds(sid * rps, rps)],
                        out_hbm.at[cid, pl.ds(sid * rps, rps)])

    return k(x, src, dst, zeros)


def _sc_degree(dst, zeros, ones):
    nw = _SC_NC * _SC_NS
    epw = E // nw
    nch = epw // _SC_CH
    npad = _SC_NPAD
    rps = npad // _SC_NS
    mesh = plsc.VectorSubcoreMesh(core_axis_name="c", subcore_axis_name="s")

    @functools.partial(
        pl.kernel, mesh=mesh,
        out_type=jax.ShapeDtypeStruct((_SC_NC, npad, D), jnp.float32),
        scratch_types=[
            pltpu.VMEM((_SC_CH,), jnp.int32),
            pltpu.VMEM((_SC_CH, D), jnp.float32),
            pltpu.VMEM_SHARED((npad, D), jnp.float32),
        ])
    def k(dst_hbm, z_hbm, o_hbm, out_hbm, didx, ones_v, acc):
        cid = lax.axis_index("c")
        sid = lax.axis_index("s")
        wid = sid * _SC_NC + cid
        pltpu.sync_copy(z_hbm.at[pl.ds(sid * rps, rps)],
                        acc.at[pl.ds(sid * rps, rps)])
        pltpu.sync_copy(o_hbm, ones_v)
        plsc.subcore_barrier()

        def body(c, carry):
            base = wid * epw + c * _SC_CH
            pltpu.sync_copy(dst_hbm.at[pl.ds(base, _SC_CH)], didx)
            pltpu.sync_copy(ones_v, acc.at[didx], add=True)
            return carry

        lax.fori_loop(0, nch, body, 0)
        plsc.subcore_barrier()
        pltpu.sync_copy(acc.at[pl.ds(sid * rps, rps)],
                        out_hbm.at[cid, pl.ds(sid * rps, rps)])

    return k(dst, zeros, ones)


def kernel(features, edge_index, W_enc, base_emb, base_labels, W1, b1, W2, b2):
    src = edge_index[0].astype(jnp.int32)
    dst = edge_index[1].astype(jnp.int32)
    zeros = jnp.zeros((_SC_NPAD, D), dtype=jnp.float32)
    dparts = _sc_degree(dst, zeros, jnp.ones((_SC_CH, D), dtype=jnp.float32))
    deg = jnp.clip(dparts[0, :N, 0] + dparts[1, :N, 0], 1.0, None)

    def hop(x):
        p = _sc_seg_sum(x, src, dst, zeros)
        return (p[0, :N] + p[1, :N]) / deg[:, None]

    h = features @ W_enc
    pre = jax.nn.relu(hop(h))

    base_pad_t = jnp.concatenate(
        [base_emb.T, jnp.zeros((D, _MPAD - M), dtype=base_emb.dtype)], axis=1)
    top_v, top_if = _knn_topk(pre, base_pad_t, _TN, _TM)
    top_i = top_if.astype(jnp.int32)
    w = jax.nn.softmax(top_v, axis=1)
    rag_embedding = jnp.einsum("nkd,nk->nd", jnp.take(base_emb, top_i, axis=0), w)
    rag_label = jnp.mean(jnp.take(base_labels, top_i, axis=0), axis=1)

    x = pre
    for _ in range(HOPS):
        x = hop(x)

    hidden = x * (1.0 - RETRIEVE_W) + rag_embedding * RETRIEVE_W
    dec = jax.nn.relu(hidden @ W1 + b1) @ W2 + b2
    decode_label = jax.nn.softmax(dec, axis=1)
    return decode_label * (1.0 - LABEL_W) + rag_label * LABEL_W
